# Initial kernel scaffold; baseline (speedup 1.0000x reference)
import jax
import jax.numpy as jnp
from jax.experimental import pallas as pl


def kernel(input, edge_index, edge_weight, W, b):
    raise NotImplementedError("write your pallas kernel here")


# speedup vs baseline: 6.7907x; 6.7907x over previous
import functools

import jax
import jax.numpy as jnp
from jax import lax
from jax.experimental import pallas as pl
from jax.experimental.pallas import tpu as pltpu
from jax.experimental.pallas import tpu_sc as plsc

_N = 10000
_E = 320000
_D = 128
_NC = 2
_NS = 16
_NW = _NC * _NS
_EPT = _E // _NW
_K = 80
_NCHUNK = _EPT // _K
_ZC = 80
_NZCH = _N // _ZC


def _sc_aggregate_body(src_hbm, dst_hbm, w_hbm, x_hbm, out_hbm,
                       src_v, dst_v, w_v, rows, acc, sem):
    c = lax.axis_index("c")
    s = lax.axis_index("s")
    wid = c * _NS + s

    pltpu.sync_copy(src_hbm.at[wid], src_v)
    pltpu.sync_copy(dst_hbm.at[wid], dst_v)
    pltpu.sync_copy(w_hbm.at[wid], w_v)

    zero16 = jnp.zeros((16,), jnp.float32)

    def zb(e, carry):
        for j in range(_D // 16):
            rows[e, pl.ds(j * 16, 16)] = zero16
        return carry

    lax.fori_loop(0, _ZC, zb, 0)
    nmine = jnp.where(s < _NZCH - (_NZCH // _NS) * _NS, _NZCH // _NS + 1,
                      _NZCH // _NS)

    def zloop(k, carry):
        i = k * _NS + s
        pltpu.sync_copy(rows, acc.at[pl.ds(i * _ZC, _ZC)])
        return carry

    lax.fori_loop(0, nmine, zloop, 0)
    plsc.subcore_barrier()

    def chunk(ci, carry):
        pltpu.async_copy(x_hbm.at[src_v.at[pl.ds(ci * _K, _K)]], rows,
                         sem).wait()

        def scale(g, c2):
            wvec = w_v[pl.ds(ci * _K + g * 16, 16)]
            for l in range(16):
                w = wvec[l]
                e = g * 16 + l
                for j in range(_D // 16):
                    sl = pl.ds(j * 16, 16)
                    rows[e, sl] = rows[e, sl] * w
            return c2

        lax.fori_loop(0, _K // 16, scale, 0)
        pltpu.sync_copy(rows, acc.at[dst_v.at[ci]], add=True)
        return carry

    lax.fori_loop(0, _NCHUNK, chunk, 0)
    plsc.subcore_barrier()

    def wloop(k, carry):
        i = k * _NS + s
        pltpu.sync_copy(acc.at[pl.ds(i * _ZC, _ZC)],
                        out_hbm.at[c, pl.ds(i * _ZC, _ZC)])
        return carry

    lax.fori_loop(0, nmine, wloop, 0)


_sc_aggregate = functools.partial(
    pl.kernel,
    mesh=plsc.VectorSubcoreMesh(core_axis_name="c", subcore_axis_name="s"),
    out_type=jax.ShapeDtypeStruct((_NC, _N, _D), jnp.float32),
    scratch_types=[
        pltpu.VMEM((_EPT,), jnp.int32),
        pltpu.VMEM((_NCHUNK, _K), jnp.int32),
        pltpu.VMEM((_EPT,), jnp.float32),
        pltpu.VMEM((_K, _D), jnp.float32),
        pltpu.VMEM_SHARED((_N, _D), jnp.float32),
        pltpu.SemaphoreType.DMA,
    ],
)(_sc_aggregate_body)


_BN = 1000


def _tc_matmul_body(p_ref, w_ref, b_ref, o_ref):
    p = p_ref[0] + p_ref[1]
    o_ref[...] = (
        jnp.dot(p, w_ref[...], preferred_element_type=jnp.float32) + b_ref[...]
    )


def _tc_matmul(partials, W, b):
    return pl.pallas_call(
        _tc_matmul_body,
        grid=(_N // _BN,),
        in_specs=[
            pl.BlockSpec((_NC, _BN, _D), lambda i: (0, i, 0)),
            pl.BlockSpec((_D, _D), lambda i: (0, 0)),
            pl.BlockSpec((1, _D), lambda i: (0, 0)),
        ],
        out_specs=pl.BlockSpec((_BN, _D), lambda i: (i, 0)),
        out_shape=jax.ShapeDtypeStruct((_N, _D), jnp.float32),
    )(partials, W, b.reshape(1, _D))


def kernel(input, edge_index, edge_weight, W, b):
    src = edge_index[1].astype(jnp.int32).reshape(_NW, _EPT)
    dst = edge_index[0].astype(jnp.int32).reshape(_NW, _NCHUNK, _K)
    w2 = edge_weight.astype(jnp.float32).reshape(_NW, _EPT)
    partials = _sc_aggregate(src, dst, w2, input)
    return _tc_matmul(partials, W, b)
